# baseline (speedup 1.0000x reference)
import jax
import jax.numpy as jnp
from jax.experimental import pallas as pl
from jax.experimental.pallas import tpu as pltpu

B = 1024
D = 128
T = 4
K = 512
M = 65536
TK = T * K

_BB = 256
_STEPS = B // _BB
_NCHUNK = 64
_CROWS = (T * M) // _NCHUNK


def _body(task_sref, write_sref, task_ref, feat_ref, cent_ref, bank_ref,
          out_ref, written, copy_sem, sc_sem):
    i = pl.program_id(0)

    @pl.when(i == 0)
    def _start_copy():
        def start(c, _):
            pltpu.make_async_copy(
                bank_ref.at[pl.ds(c * _CROWS, _CROWS), :],
                out_ref.at[pl.ds(c * _CROWS, _CROWS), :],
                copy_sem,
            ).start()
            return 0
        jax.lax.fori_loop(0, _NCHUNK, start, 0)

    feats = feat_ref[...]
    cents = cent_ref[...]
    dots = jax.lax.dot_general(
        feats, cents, (((1,), (1,)), ((), ())),
        preferred_element_type=jnp.float32)
    ones = jnp.ones((1, D), dtype=jnp.float32)
    sq = jax.lax.dot_general(
        ones, cents * cents, (((1,), (1,)), ((), ())),
        preferred_element_type=jnp.float32)
    d2 = sq - 2.0 * dots
    col = jax.lax.broadcasted_iota(jnp.int32, d2.shape, 1)
    task = task_ref[...]
    masked = jnp.where(col // K == task, d2, jnp.float32(3e38))
    mins = jnp.min(masked, axis=1, keepdims=True)
    choice = jnp.min(jnp.where(masked == mins, col, TK), axis=1, keepdims=True)
    onehot = (col == choice).astype(jnp.float32)
    sel = jax.lax.dot_general(
        onehot, cents, (((1,), (0,)), ((), ())),
        preferred_element_type=jnp.float32)
    written[pl.ds(i * _BB, _BB), :] = feats + 0.1 * sel

    @pl.when(i == _STEPS - 1)
    def _scatter():
        def drain_copy(c, _):
            pltpu.make_async_copy(
                bank_ref.at[pl.ds(0, _CROWS), :],
                out_ref.at[pl.ds(0, _CROWS), :],
                copy_sem,
            ).wait()
            return 0
        jax.lax.fori_loop(0, _NCHUNK, drain_copy, 0)

        def issue(s, _):
            flat = task_sref[s] * M + write_sref[s]
            pltpu.make_async_copy(
                written.at[pl.ds(s, 1), :],
                out_ref.at[pl.ds(flat, 1), :],
                sc_sem,
            ).start()
            return 0
        jax.lax.fori_loop(0, B, issue, 0)

        def drain(s, _):
            pltpu.make_async_copy(
                written.at[pl.ds(0, 1), :],
                out_ref.at[pl.ds(0, 1), :],
                sc_sem,
            ).wait()
            return 0
        jax.lax.fori_loop(0, B, drain, 0)


def kernel(features, feature_bank, cluster_centers, task_idx, write_idx):
    flat_centers = cluster_centers.reshape(TK, D)
    task2d = task_idx.reshape(B, 1)
    bank_flat = feature_bank.reshape(T * M, D)

    grid_spec = pltpu.PrefetchScalarGridSpec(
        num_scalar_prefetch=2,
        grid=(_STEPS,),
        in_specs=[
            pl.BlockSpec((_BB, 1), lambda i, t, w: (i, 0)),
            pl.BlockSpec((_BB, D), lambda i, t, w: (i, 0)),
            pl.BlockSpec((TK, D), lambda i, t, w: (0, 0)),
            pl.BlockSpec(memory_space=pl.ANY),
        ],
        out_specs=pl.BlockSpec(memory_space=pl.ANY),
        scratch_shapes=[
            pltpu.VMEM((B, D), jnp.float32),
            pltpu.SemaphoreType.DMA,
            pltpu.SemaphoreType.DMA,
        ],
    )
    new_bank = pl.pallas_call(
        _body,
        grid_spec=grid_spec,
        out_shape=jax.ShapeDtypeStruct((T * M, D), jnp.float32),
    )(task_idx, write_idx, task2d, features, flat_centers, bank_flat)

    return new_bank.reshape(T, M, D)


# speedup vs baseline: 41.3285x; 41.3285x over previous
import jax
import jax.numpy as jnp
from jax.experimental import pallas as pl
from jax.experimental.pallas import tpu as pltpu

B = 1024
D = 128
T = 4
K = 512
M = 65536
TK = T * K

_BB = 256
_STEPS = B // _BB


def _body(task_sref, write_sref, task_ref, feat_ref, cent_ref, bank_ref,
          out_ref, written, sem):
    del bank_ref
    i = pl.program_id(0)
    feats = feat_ref[...]
    cents = cent_ref[...]
    dots = jax.lax.dot_general(
        feats, cents, (((1,), (1,)), ((), ())),
        preferred_element_type=jnp.float32)
    ones = jnp.ones((1, D), dtype=jnp.float32)
    sq = jax.lax.dot_general(
        ones, cents * cents, (((1,), (1,)), ((), ())),
        preferred_element_type=jnp.float32)
    d2 = sq - 2.0 * dots
    col = jax.lax.broadcasted_iota(jnp.int32, d2.shape, 1)
    task = task_ref[...]
    masked = jnp.where(col // K == task, d2, jnp.float32(3e38))
    mins = jnp.min(masked, axis=1, keepdims=True)
    choice = jnp.min(jnp.where(masked == mins, col, TK), axis=1, keepdims=True)
    onehot = (col == choice).astype(jnp.float32)
    sel = jax.lax.dot_general(
        onehot, cents, (((1,), (0,)), ((), ())),
        preferred_element_type=jnp.float32)
    written[pl.ds(i * _BB, _BB), :] = feats + 0.1 * sel

    def issue(j, _):
        s = i * _BB + j
        flat = task_sref[s] * M + write_sref[s]
        pltpu.make_async_copy(
            written.at[pl.ds(s, 1), :],
            out_ref.at[pl.ds(flat, 1), :],
            sem,
        ).start()
        return 0

    jax.lax.fori_loop(0, _BB, issue, 0, unroll=8)

    @pl.when(i == _STEPS - 1)
    def _drain():
        pltpu.make_async_copy(
            written.at[...],
            out_ref.at[pl.ds(0, B), :],
            sem,
        ).wait()


def kernel(features, feature_bank, cluster_centers, task_idx, write_idx):
    flat_centers = cluster_centers.reshape(TK, D)
    task2d = task_idx.reshape(B, 1)
    bank_flat = feature_bank.reshape(T * M, D)

    grid_spec = pltpu.PrefetchScalarGridSpec(
        num_scalar_prefetch=2,
        grid=(_STEPS,),
        in_specs=[
            pl.BlockSpec((_BB, 1), lambda i, t, w: (i, 0)),
            pl.BlockSpec((_BB, D), lambda i, t, w: (i, 0)),
            pl.BlockSpec((TK, D), lambda i, t, w: (0, 0)),
            pl.BlockSpec(memory_space=pl.ANY),
        ],
        out_specs=pl.BlockSpec(memory_space=pl.ANY),
        scratch_shapes=[
            pltpu.VMEM((B, D), jnp.float32),
            pltpu.SemaphoreType.DMA,
        ],
    )
    new_bank = pl.pallas_call(
        _body,
        grid_spec=grid_spec,
        out_shape=jax.ShapeDtypeStruct((T * M, D), jnp.float32),
        input_output_aliases={5: 0},
    )(task_idx, write_idx, task2d, features, flat_centers, bank_flat)

    return new_bank.reshape(T, M, D)
